# Initial kernel scaffold; baseline (speedup 1.0000x reference)
import jax
import jax.numpy as jnp
from jax.experimental import pallas as pl


def kernel(edge_attr, W0, W1, W2):
    raise NotImplementedError("write your pallas kernel here")


# speedup vs baseline: 1.2192x; 1.2192x over previous
import functools

import jax
import jax.numpy as jnp
from jax import lax
from jax.experimental import pallas as pl
from jax.experimental.pallas import tpu as pltpu
from jax.experimental.pallas import tpu_sc as plsc

E = 320000
H = 128
D0, D1, D2 = 5, 6, 2
NT = D0 * D1 * D2
NTP = 64
NC, NS, L = 2, 16, 16
NW = NC * NS
U = 128
NU = E // U


def _sc_body(a0_hbm, a1_hbm, a2_hbm, w0_hbm, w1_hbm, w2_hbm, out_hbm, t_hbm,
             w0_v, w1_v, w2_v, t_v, a0_v, a1_v, a2_v, code_v, rows_v, gsem):
    c = lax.axis_index("c")
    s = lax.axis_index("s")
    wid = s * NC + c

    @pl.when(s == 0)
    def _build():
        pltpu.sync_copy(w0_hbm, w0_v)
        pltpu.sync_copy(w1_hbm, w1_v)
        pltpu.sync_copy(w2_hbm, w2_v)
        third = jnp.float32(1.0 / 3.0)
        for i0 in range(D0):
            for i1 in range(D1):
                for j in range(H // L):
                    sl = pl.ds(j * L, L)
                    s01 = w0_v[i0, sl] + w1_v[i1, sl]
                    r = (i0 * D1 + i1) * D2
                    t_v[r, sl] = (s01 + w2_v[0, sl]) * third
                    t_v[r + 1, sl] = (s01 + w2_v[1, sl]) * third
        for r in range(NT, NTP):
            for j in range(H // L):
                t_v[r, pl.ds(j * L, L)] = jnp.zeros((L,), jnp.float32)
        pltpu.sync_copy(t_v, t_hbm.at[pl.ds(c * NTP, NTP)])

    plsc.subcore_barrier()

    t_base = c * NTP

    def unit_body(j, carry):
        u = wid + j * NW
        pltpu.sync_copy(a0_hbm.at[pl.ds(u * U, U)], a0_v)
        pltpu.sync_copy(a1_hbm.at[pl.ds(u * U, U)], a1_v)
        pltpu.sync_copy(a2_hbm.at[pl.ds(u * U, U)], a2_v)
        for k in range(U // L):
            sl = pl.ds(k * L, L)
            code_v[sl] = (a0_v[sl] * (D1 * D2) + a1_v[sl] * D2 + a2_v[sl]) + t_base
        pltpu.async_copy(t_hbm.at[code_v], rows_v, gsem).wait()
        pltpu.sync_copy(rows_v, out_hbm.at[pl.ds(u * U, U)])
        return carry

    nj = (NU // NW) + jnp.where(wid < (NU % NW), 1, 0).astype(jnp.int32)
    lax.fori_loop(0, nj, unit_body, 0)


_launch = functools.partial(
    pl.kernel,
    out_type=(
        jax.ShapeDtypeStruct((E, H), jnp.float32),
        jax.ShapeDtypeStruct((NC * NTP, H), jnp.float32),
    ),
    mesh=plsc.VectorSubcoreMesh(core_axis_name="c", subcore_axis_name="s"),
    scratch_types=[
        pltpu.VMEM((D0, H), jnp.float32),
        pltpu.VMEM((D1, H), jnp.float32),
        pltpu.VMEM((D2, H), jnp.float32),
        pltpu.VMEM((NTP, H), jnp.float32),
        pltpu.VMEM((U,), jnp.int32),
        pltpu.VMEM((U,), jnp.int32),
        pltpu.VMEM((U,), jnp.int32),
        pltpu.VMEM((U,), jnp.int32),
        pltpu.VMEM((U, H), jnp.float32),
        pltpu.SemaphoreType.DMA,
    ],
)(_sc_body)


@jax.jit
def kernel(edge_attr, W0, W1, W2):
    ea_t = edge_attr.T
    out, _ = _launch(ea_t[0], ea_t[1], ea_t[2], W0, W1, W2)
    return out
